# baseline (speedup 1.0000x reference)
import functools

import jax
import jax.numpy as jnp
from jax import lax
from jax.experimental import pallas as pl
from jax.experimental.pallas import tpu as pltpu
from jax.experimental.pallas import tpu_sc as plsc

_B = 8
_N = 5000
_NUM_DET = 3
_IOU_THRESH = 0.5
_L = 16
_NFULL = _N // _L
_BIG_I32 = 2**31 - 1


def _vgather(x, idx):
    dnums = lax.GatherDimensionNumbers(
        offset_dims=(), collapsed_slice_dims=(0,), start_index_map=(0,))
    return lax.gather(x, idx[:, None], dnums, (1,),
                      mode=lax.GatherScatterMode.PROMISE_IN_BOUNDS)


def _butterfly(x, op, lane):
    for sh in (8, 4, 2, 1):
        x = op(x, _vgather(x, lane ^ sh))
    return x


def _nms_body(scores_hbm, boxesf_hbm, classes_hbm, out_hbm,
              s_v, bx_v, x1_v, y1_v, x2_v, y2_v, cls_v, out_v, pack_v,
              sh_out, sem):
    wid = lax.axis_index("s")
    lane = lax.iota(jnp.int32, _L)

    @pl.when(wid < _B)
    def _():
        b = wid
        copies = [
            pltpu.async_copy(scores_hbm.at[b], s_v, sem),
            pltpu.async_copy(boxesf_hbm.at[b], bx_v, sem),
            pltpu.async_copy(classes_hbm.at[b], cls_v, sem),
        ]
        for cp in copies:
            cp.wait()

        neg_inf = jnp.float32(-jnp.inf)
        bv0 = jnp.full((_L,), neg_inf, jnp.float32)
        bi0 = jnp.zeros((_L,), jnp.int32)

        def pass_a(off, carry):
            bv, bi = carry
            idx = off + lane
            base4 = idx * 4
            sl = pl.ds(off, _L)
            x1_v[sl] = plsc.load_gather(bx_v, [base4])
            y1_v[sl] = plsc.load_gather(bx_v, [base4 + 1])
            x2_v[sl] = plsc.load_gather(bx_v, [base4 + 2])
            y2_v[sl] = plsc.load_gather(bx_v, [base4 + 3])
            sv = s_v[sl]
            cond = sv > bv
            return jnp.where(cond, sv, bv), jnp.where(cond, idx, bi)

        def winner_of(carry):
            bv, bi = carry
            m = _butterfly(bv, jnp.maximum, lane)
            cand = jnp.where(bv == m, bi, jnp.int32(_BIG_I32))
            return _butterfly(cand, jnp.minimum, lane)

        def winner_data(wv):
            wx1 = plsc.load_gather(x1_v, [wv])
            wy1 = plsc.load_gather(y1_v, [wv])
            wx2 = plsc.load_gather(x2_v, [wv])
            wy2 = plsc.load_gather(y2_v, [wv])
            wcls = plsc.load_gather(cls_v, [wv])
            warea = (jnp.maximum(wx2 - wx1, jnp.float32(0.0)) *
                     jnp.maximum(wy2 - wy1, jnp.float32(0.0)))
            return wx1, wy1, wx2, wy2, wcls, warea

        def fused_body(wd, off, carry):
            wx1, wy1, wx2, wy2, wcls, warea = wd
            bv, bi = carry
            sl = pl.ds(off, _L)
            x1c = x1_v[sl]
            y1c = y1_v[sl]
            x2c = x2_v[sl]
            y2c = y2_v[sl]
            ix1 = jnp.maximum(wx1, x1c)
            iy1 = jnp.maximum(wy1, y1c)
            ix2 = jnp.minimum(wx2, x2c)
            iy2 = jnp.minimum(wy2, y2c)
            inter = (jnp.maximum(ix2 - ix1, jnp.float32(0.0)) *
                     jnp.maximum(iy2 - iy1, jnp.float32(0.0)))
            area_b = (jnp.maximum(x2c - x1c, jnp.float32(0.0)) *
                      jnp.maximum(y2c - y1c, jnp.float32(0.0)))
            iou = inter / jnp.maximum(warea + area_b - inter, jnp.float32(1e-9))
            supp = (iou > jnp.float32(_IOU_THRESH)) & (cls_v[sl] == wcls)
            sv = jnp.where(supp, neg_inf, s_v[sl])
            s_v[sl] = sv
            idx = off + lane
            cond = sv > bv
            return jnp.where(cond, sv, bv), jnp.where(cond, idx, bi)

        def full_sweep(body, carry):
            carry = lax.fori_loop(
                0, _NFULL, lambda i, c: body(i * _L, c), carry, unroll=4)
            return body(_N - _L, carry)

        out_v[...] = jnp.zeros((_L,), jnp.int32)
        carry = full_sweep(pass_a, (bv0, bi0))
        for d in range(_NUM_DET):
            wv = winner_of(carry)
            out_v[...] = jnp.where(lane == d, wv, out_v[...])
            if d < _NUM_DET - 1:
                wd = winner_data(wv)
                carry = full_sweep(functools.partial(fused_body, wd),
                                   (bv0, bi0))
        pltpu.sync_copy(out_v, sh_out.at[pl.ds(b * _L, _L)])

    plsc.subcore_barrier()

    @pl.when(wid == 0)
    def _():
        pltpu.sync_copy(sh_out, pack_v.at[pl.ds(0, _B * _L)])
        for chunk in range(2):
            k = chunk * _L + lane
            q = (k * 21846) >> 16
            src = jnp.minimum(q * _L + (k - q * 3), _B * _L - 1)
            pack_v[pl.ds(_B * _L + chunk * _L, _L)] = plsc.load_gather(
                pack_v, [src])
        pltpu.sync_copy(pack_v.at[pl.ds(_B * _L, _B * _NUM_DET)], out_hbm)


@jax.jit
def _nms_sc(scores, boxesf, classes):
    mesh = plsc.VectorSubcoreMesh(core_axis_name="c", subcore_axis_name="s",
                                  num_cores=1)
    f = pl.kernel(
        _nms_body,
        out_type=jax.ShapeDtypeStruct((_B * _NUM_DET,), jnp.int32),
        mesh=mesh,
        scratch_types=[
            pltpu.VMEM((_N,), jnp.float32),
            pltpu.VMEM((4 * _N,), jnp.float32),
            pltpu.VMEM((_N,), jnp.float32),
            pltpu.VMEM((_N,), jnp.float32),
            pltpu.VMEM((_N,), jnp.float32),
            pltpu.VMEM((_N,), jnp.float32),
            pltpu.VMEM((_N,), jnp.int32),
            pltpu.VMEM((_L,), jnp.int32),
            pltpu.VMEM((_B * _L + 2 * _L,), jnp.int32),
            pltpu.VMEM_SHARED((_B * _L,), jnp.int32),
            pltpu.SemaphoreType.DMA,
        ],
        compiler_params=pltpu.CompilerParams(needs_layout_passes=False),
    )
    return f(scores, boxesf, classes)


def kernel(scores, boxes, classes):
    boxesf = boxes.reshape(_B, 4 * _N)
    out = _nms_sc(scores, boxesf, classes)
    return out.reshape(_B, _NUM_DET)


# speedup vs baseline: 1.6260x; 1.6260x over previous
import functools

import jax
import jax.numpy as jnp
from jax import lax
from jax.experimental import pallas as pl
from jax.experimental.pallas import tpu as pltpu
from jax.experimental.pallas import tpu_sc as plsc

_B = 8
_N = 5000
_NUM_DET = 3
_IOU_THRESH = 0.5
_L = 16
_NFULL = _N // _L
_BIG_I32 = 2**31 - 1


def _vgather(x, idx):
    dnums = lax.GatherDimensionNumbers(
        offset_dims=(), collapsed_slice_dims=(0,), start_index_map=(0,))
    return lax.gather(x, idx[:, None], dnums, (1,),
                      mode=lax.GatherScatterMode.PROMISE_IN_BOUNDS)


def _butterfly(x, op, lane):
    for sh in (8, 4, 2, 1):
        x = op(x, _vgather(x, lane ^ sh))
    return x


def _nms_body(scores_hbm, boxest_hbm, classes_hbm, out_hbm,
              s_v, x1_v, y1_v, x2_v, y2_v, cls_v, out_v, pack_v, out_2d,
              sh_out, sem):
    wid = lax.axis_index("s")
    lane = lax.iota(jnp.int32, _L)

    @pl.when(wid < _B)
    def _():
        b = wid
        copies = [
            pltpu.async_copy(scores_hbm.at[b], s_v, sem),
            pltpu.async_copy(boxest_hbm.at[b, 0], x1_v, sem),
            pltpu.async_copy(boxest_hbm.at[b, 1], y1_v, sem),
            pltpu.async_copy(boxest_hbm.at[b, 2], x2_v, sem),
            pltpu.async_copy(boxest_hbm.at[b, 3], y2_v, sem),
            pltpu.async_copy(classes_hbm.at[b], cls_v, sem),
        ]
        for cp in copies:
            cp.wait()

        neg_inf = jnp.float32(-jnp.inf)
        bv0 = jnp.full((_L,), neg_inf, jnp.float32)
        bi0 = jnp.zeros((_L,), jnp.int32)

        def pass_a(off, carry):
            bv, bi = carry
            sv = s_v[pl.ds(off, _L)]
            idx = off + lane
            cond = sv > bv
            return jnp.where(cond, sv, bv), jnp.where(cond, idx, bi)

        def winner_of(carry):
            bv, bi = carry
            m = _butterfly(bv, jnp.maximum, lane)
            cand = jnp.where(bv == m, bi, jnp.int32(_BIG_I32))
            return _butterfly(cand, jnp.minimum, lane)

        def winner_data(wv):
            wx1 = plsc.load_gather(x1_v, [wv])
            wy1 = plsc.load_gather(y1_v, [wv])
            wx2 = plsc.load_gather(x2_v, [wv])
            wy2 = plsc.load_gather(y2_v, [wv])
            wcls = plsc.load_gather(cls_v, [wv])
            warea = (jnp.maximum(wx2 - wx1, jnp.float32(0.0)) *
                     jnp.maximum(wy2 - wy1, jnp.float32(0.0)))
            return wx1, wy1, wx2, wy2, wcls, warea

        def fused_body(wd, off, carry):
            wx1, wy1, wx2, wy2, wcls, warea = wd
            bv, bi = carry
            sl = pl.ds(off, _L)
            x1c = x1_v[sl]
            y1c = y1_v[sl]
            x2c = x2_v[sl]
            y2c = y2_v[sl]
            ix1 = jnp.maximum(wx1, x1c)
            iy1 = jnp.maximum(wy1, y1c)
            ix2 = jnp.minimum(wx2, x2c)
            iy2 = jnp.minimum(wy2, y2c)
            inter = (jnp.maximum(ix2 - ix1, jnp.float32(0.0)) *
                     jnp.maximum(iy2 - iy1, jnp.float32(0.0)))
            area_b = (jnp.maximum(x2c - x1c, jnp.float32(0.0)) *
                      jnp.maximum(y2c - y1c, jnp.float32(0.0)))
            iou = inter / jnp.maximum(warea + area_b - inter, jnp.float32(1e-9))
            supp = (iou > jnp.float32(_IOU_THRESH)) & (cls_v[sl] == wcls)
            sv = jnp.where(supp, neg_inf, s_v[sl])
            s_v[sl] = sv
            idx = off + lane
            cond = sv > bv
            return jnp.where(cond, sv, bv), jnp.where(cond, idx, bi)

        def full_sweep(body, carry):
            carry = lax.fori_loop(
                0, _NFULL, lambda i, c: body(i * _L, c), carry, unroll=4)
            return body(_N - _L, carry)

        out_v[...] = jnp.zeros((_L,), jnp.int32)
        carry = full_sweep(pass_a, (bv0, bi0))
        for d in range(_NUM_DET):
            wv = winner_of(carry)
            out_v[...] = jnp.where(lane == d, wv, out_v[...])
            if d < _NUM_DET - 1:
                wd = winner_data(wv)
                carry = full_sweep(functools.partial(fused_body, wd),
                                   (bv0, bi0))
        pltpu.sync_copy(out_v, sh_out.at[pl.ds(b * _L, _L)])

    plsc.subcore_barrier()

    @pl.when(wid == 0)
    def _():
        pltpu.sync_copy(sh_out, pack_v)
        for chunk in range(2):
            k = chunk * _L + lane
            q = (k * 21846) >> 16
            r = k - q * 3
            src = jnp.minimum(q * _L + r, _B * _L - 1)
            vals = plsc.load_gather(pack_v, [src])
            plsc.store_scatter(out_2d, [jnp.minimum(q, _B - 1), r], vals,
                               mask=k < _B * _NUM_DET)
        pltpu.sync_copy(out_2d, out_hbm)


@jax.jit
def _nms_sc(scores, boxest, classes):
    mesh = plsc.VectorSubcoreMesh(core_axis_name="c", subcore_axis_name="s",
                                  num_cores=1)
    f = pl.kernel(
        _nms_body,
        out_type=jax.ShapeDtypeStruct((_B, _NUM_DET), jnp.int32),
        mesh=mesh,
        scratch_types=[
            pltpu.VMEM((_N,), jnp.float32),
            pltpu.VMEM((_N,), jnp.float32),
            pltpu.VMEM((_N,), jnp.float32),
            pltpu.VMEM((_N,), jnp.float32),
            pltpu.VMEM((_N,), jnp.float32),
            pltpu.VMEM((_N,), jnp.int32),
            pltpu.VMEM((_L,), jnp.int32),
            pltpu.VMEM((_B * _L,), jnp.int32),
            pltpu.VMEM((_B, _NUM_DET), jnp.int32),
            pltpu.VMEM_SHARED((_B * _L,), jnp.int32),
            pltpu.SemaphoreType.DMA,
        ],
        compiler_params=pltpu.CompilerParams(needs_layout_passes=False),
    )
    return f(scores, boxest, classes)


def kernel(scores, boxes, classes):
    boxest = boxes.transpose(0, 2, 1)
    return _nms_sc(scores, boxest, classes)
